# baseline (speedup 1.0000x reference)
import functools

import jax
import jax.numpy as jnp
from jax import lax
from jax.experimental import pallas as pl
from jax.experimental.pallas import tpu as pltpu
from jax.experimental.pallas import tpu_sc as plsc

N = 50000
E = 800000
EPAD = 819200
C = 64
CH = 32
NELEM = 10
NB = 8
CUTOFF = 5.0
AVG_NEIGH = 16.0

BB = 512
NBATCH = EPAD // BB
NSPLIT = BB // 128
EB = 256
ESPLIT = EB // 128
ENBATCH = EPAD // EB
NSUB = 16
NROW = N // NSUB

BN = 2000
GRID_N = N // BN

TBL = 16384
BT = 2048
SCALE_R = TBL / CUTOFF

_PI = 3.14159265358979
_MAGIC = 0x5F3759DF


def _silu(x):
    return x * jax.nn.sigmoid(x)


def _rsqrt_newton(r2):
    i = plsc.bitcast(r2, jnp.int32)
    y = plsc.bitcast(_MAGIC - lax.shift_right_logical(i, 1), jnp.float32)
    for _ in range(3):
        y = y * (1.5 - 0.5 * r2 * y * y)
    return y


def _sin_cos(u):
    u2 = u * u
    s = u * (1.0 + u2 * (-1.0 / 6 + u2 * (1.0 / 120 + u2 * (-1.0 / 5040
        + u2 * (1.0 / 362880 - u2 * (1.0 / 39916800))))))
    c = 1.0 + u2 * (-0.5 + u2 * (1.0 / 24 + u2 * (-1.0 / 720
        + u2 * (1.0 / 40320 + u2 * (-1.0 / 3628800 + u2 * (1.0 / 479001600))))))
    return s, c


def _rb_sc(pos_pad, src2d, dst2d):
    mesh = plsc.VectorSubcoreMesh(core_axis_name="c", subcore_axis_name="s")

    @functools.partial(
        pl.kernel, mesh=mesh,
        compiler_params=pltpu.CompilerParams(needs_layout_passes=False, use_tc_tiling_on_sc=False),
        out_type=jax.ShapeDtypeStruct((EPAD // 128, 128), jnp.int32),
        scratch_types=[
            pltpu.VMEM((NSPLIT, 128), jnp.int32),
            pltpu.VMEM((NSPLIT, 128), jnp.int32),
            pltpu.VMEM((BB, 16), jnp.float32),
            pltpu.VMEM((BB, 16), jnp.float32),
            pltpu.VMEM((NSPLIT, 128), jnp.int32),
            pltpu.SemaphoreType.DMA,
        ],
    )
    def k(pos_ref, src_ref, dst_ref, ridx_ref, srcv, dstv, ps, pd, idxb, sem):
        cid = lax.axis_index("c")
        sid = lax.axis_index("s")
        wid = sid * 2 + cid

        def batch_body(kk, carry):
            bi = kk * 32 + wid
            pltpu.sync_copy(src_ref.at[pl.ds(bi * NSPLIT, NSPLIT)], srcv)
            pltpu.sync_copy(dst_ref.at[pl.ds(bi * NSPLIT, NSPLIT)], dstv)
            cps = []
            for j in range(NSPLIT):
                cps.append(pltpu.async_copy(
                    pos_ref.at[srcv.at[j]], ps.at[pl.ds(j * 128, 128)], sem))
                cps.append(pltpu.async_copy(
                    pos_ref.at[dstv.at[j]], pd.at[pl.ds(j * 128, 128)], sem))
            for cp in cps:
                cp.wait()
            base = bi * BB

            for g in range(BB // 16):
                e0 = g * 16
                erow = lax.broadcasted_iota(jnp.int32, (16,), 0) + e0
                col = jnp.zeros((16,), jnp.int32)
                xs = plsc.load_gather(ps, [erow, col])
                ys = plsc.load_gather(ps, [erow, col + 1])
                zs = plsc.load_gather(ps, [erow, col + 2])
                xd = plsc.load_gather(pd, [erow, col])
                yd = plsc.load_gather(pd, [erow, col + 1])
                zd = plsc.load_gather(pd, [erow, col + 2])
                dx = xd - xs
                dy = yd - ys
                dz = zd - zs
                r2 = dx * dx + dy * dy + dz * dz + 1e-12
                y = _rsqrt_newton(r2)
                r = r2 * y
                idx = jnp.minimum((r * SCALE_R).astype(jnp.int32), TBL - 1)
                valid = (erow + base) < E
                idx = jnp.where(valid, idx, TBL - 1)
                idxb[g // 8, pl.ds((g % 8) * 16, 16)] = idx
            pltpu.sync_copy(idxb, ridx_ref.at[pl.ds(bi * NSPLIT, NSPLIT)])
            return carry

        lax.fori_loop(0, NBATCH // 32, batch_body, 0)

    return k(pos_pad, src2d, dst2d)


def _edge_sc(hcat, pk2d, table2, zrows):
    mesh = plsc.VectorSubcoreMesh(core_axis_name="c", subcore_axis_name="s")

    @functools.partial(
        pl.kernel, mesh=mesh,
        compiler_params=pltpu.CompilerParams(needs_layout_passes=False, use_tc_tiling_on_sc=False),
        out_type=[
            jax.ShapeDtypeStruct((N, CH), jnp.float32),
            jax.ShapeDtypeStruct((N, CH), jnp.float32),
        ],
        scratch_types=[
            pltpu.VMEM((3 * NSPLIT, 128), jnp.int32),
            pltpu.VMEM((BB, CH), jnp.float32),
            pltpu.VMEM((2, 128, CH), jnp.float32),
            pltpu.VMEM_SHARED((N, CH), jnp.float32),
            pltpu.SemaphoreType.DMA,
            pltpu.SemaphoreType.DMA,
            pltpu.SemaphoreType.DMA,
            pltpu.SemaphoreType.DMA,
        ],
    )
    def k(hcat_ref, pk_ref, tbl_ref, z_ref,
          alo_ref, ahi_ref, pidx, hs, radb, aggs,
          sem_h, sem_r0, sem_r1, sem_s):
        cid = lax.axis_index("c")
        sid = lax.axis_index("s")
        r0 = sid * NROW
        pltpu.sync_copy(z_ref.at[pl.ds(r0, NROW)], aggs.at[pl.ds(r0, NROW)])
        plsc.subcore_barrier()
        hoff = cid * N
        toff = cid * TBL
        sem_r = (sem_r0, sem_r1)

        def batch_body(kk, carry):
            bi = kk * NSUB + sid
            pltpu.sync_copy(pk_ref.at[pl.ds(bi * 3 * NSPLIT, 3 * NSPLIT)],
                            pidx)
            for j in range(NSPLIT):
                for v in range(8):
                    pidx[j, pl.ds(v * 16, 16)] = (
                        pidx[j, pl.ds(v * 16, 16)] + hoff)
                    pidx[2 * NSPLIT + j, pl.ds(v * 16, 16)] = (
                        pidx[2 * NSPLIT + j, pl.ds(v * 16, 16)] + toff)
            h_cps = [pltpu.async_copy(hcat_ref.at[pidx.at[j]],
                                      hs.at[pl.ds(j * 128, 128)], sem_h)
                     for j in range(NSPLIT)]
            r_cps = [None] * NSPLIT
            r_cps[0] = pltpu.async_copy(tbl_ref.at[pidx.at[2 * NSPLIT]],
                                        radb.at[0], sem_r0)
            for cp in h_cps:
                cp.wait()
            sc_cps = []
            for j in range(NSPLIT):
                if j + 1 < NSPLIT:
                    r_cps[j + 1] = pltpu.async_copy(
                        tbl_ref.at[pidx.at[2 * NSPLIT + j + 1]],
                        radb.at[(j + 1) % 2], sem_r[(j + 1) % 2])
                r_cps[j].wait()

                def mul_body(e, carry2, _j=j):
                    e2 = _j * 128 + e
                    rs = _j % 2
                    hs[e2, pl.ds(0, 16)] = (
                        hs[e2, pl.ds(0, 16)] * radb[rs, e, pl.ds(0, 16)])
                    hs[e2, pl.ds(16, 16)] = (
                        hs[e2, pl.ds(16, 16)] * radb[rs, e, pl.ds(16, 16)])
                    return carry2

                lax.fori_loop(0, 128, mul_body, 0, unroll=2)
                sc_cps.append(pltpu.async_copy(
                    hs.at[pl.ds(j * 128, 128)],
                    aggs.at[pidx.at[NSPLIT + j]], sem_s, add=True))
            for cp in sc_cps:
                cp.wait()
            return carry

        lax.fori_loop(0, NBATCH // NSUB, batch_body, 0)
        plsc.subcore_barrier()

        @pl.when(cid == 0)
        def _():
            pltpu.sync_copy(aggs.at[pl.ds(r0, NROW)], alo_ref.at[pl.ds(r0, NROW)])

        @pl.when(cid == 1)
        def _():
            pltpu.sync_copy(aggs.at[pl.ds(r0, NROW)], ahi_ref.at[pl.ds(r0, NROW)])

    return k(hcat, pk2d, table2, zrows)


def _table_body(wrad_ref, out_ref):
    i = pl.program_id(0)
    t = (lax.broadcasted_iota(jnp.int32, (BT, 1), 0) + i * BT
         ).astype(jnp.float32)
    r = (t + 0.5) * (CUTOFF / TBL)
    rs = r + 1e-9
    theta = r * (_PI / CUTOFF)
    su, cu = _sin_cos(theta - _PI / 2)
    s_n = cu
    two_c = -(su + su)
    x = r * (1.0 / CUTOFF)
    x2 = x * x
    x5 = x2 * x2 * x
    env = 1.0 + x5 * (-21.0 + 35.0 * x - 15.0 * x2)
    coef = 0.6324555320336759 * env / rs
    cols = []
    s_prev = jnp.zeros((BT, 1), jnp.float32)
    for _ in range(NB):
        cols.append(coef * s_n)
        s_next = two_c * s_n - s_prev
        s_prev = s_n
        s_n = s_next
    rb = jnp.concatenate(cols, axis=1)
    g = jnp.dot(rb, wrad_ref[...], preferred_element_type=jnp.float32)
    rad = _silu(g)
    out_ref[0] = rad[:, :CH]
    out_ref[1] = rad[:, CH:]


def _tc_table(W_rad):
    return pl.pallas_call(
        _table_body,
        grid=(TBL // BT,),
        in_specs=[pl.BlockSpec((NB, C), lambda i: (0, 0))],
        out_specs=pl.BlockSpec((2, BT, CH), lambda i: (0, i, 0)),
        out_shape=jax.ShapeDtypeStruct((2, TBL, CH), jnp.float32),
    )(W_rad)


def _embed_body(sp_ref, we_ref, h2_ref):
    sp = sp_ref[0, 0, :]
    oh = (sp[:, None] == lax.broadcasted_iota(jnp.int32, (BN, NELEM), 1)
          ).astype(jnp.float32)
    h0 = jnp.dot(oh, we_ref[...], preferred_element_type=jnp.float32)
    h2_ref[0] = h0[:, :CH]
    h2_ref[1] = h0[:, CH:]


def _tc_embed(species_r, W_embed):
    return pl.pallas_call(
        _embed_body,
        grid=(GRID_N,),
        in_specs=[
            pl.BlockSpec((1, 1, BN), lambda i: (i, 0, 0)),
            pl.BlockSpec((NELEM, C), lambda i: (0, 0)),
        ],
        out_specs=pl.BlockSpec((2, BN, CH), lambda i: (0, i, 0)),
        out_shape=jax.ShapeDtypeStruct((2, N, CH), jnp.float32),
    )(species_r, W_embed)


def _layer_body(alo_ref, ahi_ref, h2_ref, wi_ref, ws_ref, wr_ref,
                n2_ref, e_ref):
    agg = jnp.concatenate([alo_ref[...], ahi_ref[...]], axis=1) * (1.0 / AVG_NEIGH)
    h = jnp.concatenate([h2_ref[0], h2_ref[1]], axis=1)
    z = (jnp.dot(agg, wi_ref[...], preferred_element_type=jnp.float32)
         + jnp.dot(h, ws_ref[...], preferred_element_type=jnp.float32))
    hn = _silu(z)
    n2_ref[0] = hn[:, :CH]
    n2_ref[1] = hn[:, CH:]
    e_ref[...] = jnp.dot(hn, wr_ref[...], preferred_element_type=jnp.float32)


def _tc_layer(agg_lo, agg_hi, h2, W_inter, W_skip, w_read):
    return pl.pallas_call(
        _layer_body,
        grid=(GRID_N,),
        in_specs=[
            pl.BlockSpec((BN, CH), lambda i: (i, 0)),
            pl.BlockSpec((BN, CH), lambda i: (i, 0)),
            pl.BlockSpec((2, BN, CH), lambda i: (0, i, 0)),
            pl.BlockSpec((C, C), lambda i: (0, 0)),
            pl.BlockSpec((C, C), lambda i: (0, 0)),
            pl.BlockSpec((C, 1), lambda i: (0, 0)),
        ],
        out_specs=[
            pl.BlockSpec((2, BN, CH), lambda i: (0, i, 0)),
            pl.BlockSpec((BN, 1), lambda i: (i, 0)),
        ],
        out_shape=[
            jax.ShapeDtypeStruct((2, N, CH), jnp.float32),
            jax.ShapeDtypeStruct((N, 1), jnp.float32),
        ],
    )(agg_lo, agg_hi, h2, W_inter, W_skip, w_read)


def _final_body(alo_ref, ahi_ref, h2_ref, wi_ref, ws_ref, wr_ref,
                e0_ref, sp_ref, ae_ref, sc_ref, sh_ref, out_ref):
    agg = jnp.concatenate([alo_ref[...], ahi_ref[...]], axis=1) * (1.0 / AVG_NEIGH)
    h = jnp.concatenate([h2_ref[0], h2_ref[1]], axis=1)
    z = (jnp.dot(agg, wi_ref[...], preferred_element_type=jnp.float32)
         + jnp.dot(h, ws_ref[...], preferred_element_type=jnp.float32))
    hn = _silu(z)
    e1 = jnp.dot(hn, wr_ref[...], preferred_element_type=jnp.float32)
    sp = sp_ref[0, 0, :]
    oh = (sp[:, None] == lax.broadcasted_iota(jnp.int32, (BN, NELEM), 1)
          ).astype(jnp.float32)
    e_base = jnp.dot(oh, ae_ref[...], preferred_element_type=jnp.float32)
    e_read = e0_ref[...] + e1
    out_ref[...] = e_base + sc_ref[0, 0] * e_read + sh_ref[0, 0]


def _tc_final(agg_lo, agg_hi, h2, W_inter, W_skip, w_read, e0,
              species_r, ae2, scale2, shift2):
    return pl.pallas_call(
        _final_body,
        grid=(GRID_N,),
        in_specs=[
            pl.BlockSpec((BN, CH), lambda i: (i, 0)),
            pl.BlockSpec((BN, CH), lambda i: (i, 0)),
            pl.BlockSpec((2, BN, CH), lambda i: (0, i, 0)),
            pl.BlockSpec((C, C), lambda i: (0, 0)),
            pl.BlockSpec((C, C), lambda i: (0, 0)),
            pl.BlockSpec((C, 1), lambda i: (0, 0)),
            pl.BlockSpec((BN, 1), lambda i: (i, 0)),
            pl.BlockSpec((1, 1, BN), lambda i: (i, 0, 0)),
            pl.BlockSpec((NELEM, 1), lambda i: (0, 0)),
            pl.BlockSpec((1, 1), lambda i: (0, 0)),
            pl.BlockSpec((1, 1), lambda i: (0, 0)),
        ],
        out_specs=pl.BlockSpec((BN, 1), lambda i: (i, 0)),
        out_shape=jax.ShapeDtypeStruct((N, 1), jnp.float32),
    )(agg_lo, agg_hi, h2, W_inter, W_skip, w_read, e0,
      species_r, ae2, scale2, shift2)


def kernel(positions, species, edge_index, W_embed, W_rad0, W_rad1,
           W_inter0, W_inter1, W_skip0, W_skip1, w_read0, w_read1,
           atomic_energies, scale, shift):
    src2d = jnp.pad(edge_index[0].astype(jnp.int32),
                    (0, EPAD - E)).reshape(EPAD // 128, 128)
    dst2d = jnp.pad(edge_index[1].astype(jnp.int32),
                    (0, EPAD - E)).reshape(EPAD // 128, 128)
    pos_pad = jnp.pad(positions, ((0, 0), (0, 13)))
    zrows = jnp.zeros((N, CH), jnp.float32)
    species_r = species.astype(jnp.int32).reshape(GRID_N, 1, BN)
    ae2 = atomic_energies.reshape(NELEM, 1)
    scale2 = scale.reshape(1, 1)
    shift2 = shift.reshape(1, 1)

    ridx = _rb_sc(pos_pad, src2d, dst2d)
    pk2d = jnp.concatenate(
        [src2d.reshape(NBATCH, NSPLIT, 128),
         dst2d.reshape(NBATCH, NSPLIT, 128),
         ridx.reshape(NBATCH, NSPLIT, 128)], axis=1,
    ).reshape(NBATCH * 3 * NSPLIT, 128)
    tbl0 = _tc_table(W_rad0).reshape(2 * TBL, CH)
    tbl1 = _tc_table(W_rad1).reshape(2 * TBL, CH)
    h2 = _tc_embed(species_r, W_embed)

    a0lo, a0hi = _edge_sc(h2.reshape(2 * N, CH), pk2d, tbl0, zrows)
    h2n, e0 = _tc_layer(a0lo, a0hi, h2, W_inter0, W_skip0, w_read0)

    a1lo, a1hi = _edge_sc(h2n.reshape(2 * N, CH), pk2d, tbl1, zrows)
    out = _tc_final(a1lo, a1hi, h2n, W_inter1, W_skip1, w_read1, e0,
                    species_r, ae2, scale2, shift2)
    return out[:, 0]


# speedup vs baseline: 1.1900x; 1.1900x over previous
import functools

import jax
import jax.numpy as jnp
from jax import lax
from jax.experimental import pallas as pl
from jax.experimental.pallas import tpu as pltpu
from jax.experimental.pallas import tpu_sc as plsc

N = 50000
E = 800000
EPAD = 819200
C = 64
CH = 32
NELEM = 10
NB = 8
CUTOFF = 5.0
AVG_NEIGH = 16.0

BB = 512
NBATCH = EPAD // BB
NSPLIT = BB // 128
EB = 256
ESPLIT = EB // 128
ENBATCH = EPAD // EB
NSUB = 16
NROW = N // NSUB

BN = 2000
GRID_N = N // BN

TBL = 16384
BT = 2048
SCALE_R = TBL / CUTOFF

_PI = 3.14159265358979
_MAGIC = 0x5F3759DF


def _silu(x):
    return x * jax.nn.sigmoid(x)


def _rsqrt_newton(r2):
    i = plsc.bitcast(r2, jnp.int32)
    y = plsc.bitcast(_MAGIC - lax.shift_right_logical(i, 1), jnp.float32)
    for _ in range(3):
        y = y * (1.5 - 0.5 * r2 * y * y)
    return y


def _sin_cos(u):
    u2 = u * u
    s = u * (1.0 + u2 * (-1.0 / 6 + u2 * (1.0 / 120 + u2 * (-1.0 / 5040
        + u2 * (1.0 / 362880 - u2 * (1.0 / 39916800))))))
    c = 1.0 + u2 * (-0.5 + u2 * (1.0 / 24 + u2 * (-1.0 / 720
        + u2 * (1.0 / 40320 + u2 * (-1.0 / 3628800 + u2 * (1.0 / 479001600))))))
    return s, c


def _rb_sc(pos_pad, src2d, dst2d):
    mesh = plsc.VectorSubcoreMesh(core_axis_name="c", subcore_axis_name="s")

    @functools.partial(
        pl.kernel, mesh=mesh,
        compiler_params=pltpu.CompilerParams(needs_layout_passes=False, use_tc_tiling_on_sc=False),
        out_type=jax.ShapeDtypeStruct((EPAD // 128, 128), jnp.int32),
        scratch_types=[
            pltpu.VMEM((NSPLIT, 128), jnp.int32),
            pltpu.VMEM((NSPLIT, 128), jnp.int32),
            pltpu.VMEM((BB, 16), jnp.float32),
            pltpu.VMEM((BB, 16), jnp.float32),
            pltpu.VMEM((NSPLIT, 128), jnp.int32),
            pltpu.SemaphoreType.DMA,
        ],
    )
    def k(pos_ref, src_ref, dst_ref, ridx_ref, srcv, dstv, ps, pd, idxb, sem):
        cid = lax.axis_index("c")
        sid = lax.axis_index("s")
        wid = sid * 2 + cid

        def batch_body(kk, carry):
            bi = kk * 32 + wid
            pltpu.sync_copy(src_ref.at[pl.ds(bi * NSPLIT, NSPLIT)], srcv)
            pltpu.sync_copy(dst_ref.at[pl.ds(bi * NSPLIT, NSPLIT)], dstv)
            cps = []
            for j in range(NSPLIT):
                cps.append(pltpu.async_copy(
                    pos_ref.at[srcv.at[j]], ps.at[pl.ds(j * 128, 128)], sem))
                cps.append(pltpu.async_copy(
                    pos_ref.at[dstv.at[j]], pd.at[pl.ds(j * 128, 128)], sem))
            for cp in cps:
                cp.wait()
            base = bi * BB

            for g in range(BB // 16):
                e0 = g * 16
                erow = lax.broadcasted_iota(jnp.int32, (16,), 0) + e0
                col = jnp.zeros((16,), jnp.int32)
                xs = plsc.load_gather(ps, [erow, col])
                ys = plsc.load_gather(ps, [erow, col + 1])
                zs = plsc.load_gather(ps, [erow, col + 2])
                xd = plsc.load_gather(pd, [erow, col])
                yd = plsc.load_gather(pd, [erow, col + 1])
                zd = plsc.load_gather(pd, [erow, col + 2])
                dx = xd - xs
                dy = yd - ys
                dz = zd - zs
                r2 = dx * dx + dy * dy + dz * dz + 1e-12
                y = _rsqrt_newton(r2)
                r = r2 * y
                idx = jnp.minimum((r * SCALE_R).astype(jnp.int32), TBL - 1)
                valid = (erow + base) < E
                idx = jnp.where(valid, idx, TBL - 1)
                idxb[g // 8, pl.ds((g % 8) * 16, 16)] = idx
            pltpu.sync_copy(idxb, ridx_ref.at[pl.ds(bi * NSPLIT, NSPLIT)])
            return carry

        lax.fori_loop(0, NBATCH // 32, batch_body, 0)

    return k(pos_pad, src2d, dst2d)


def _edge_sc(hcat, pk2d, table2, zrows):
    mesh = plsc.VectorSubcoreMesh(core_axis_name="c", subcore_axis_name="s")

    @functools.partial(
        pl.kernel, mesh=mesh,
        compiler_params=pltpu.CompilerParams(needs_layout_passes=False, use_tc_tiling_on_sc=False),
        out_type=[
            jax.ShapeDtypeStruct((N, CH), jnp.float32),
            jax.ShapeDtypeStruct((N, CH), jnp.float32),
        ],
        scratch_types=[
            pltpu.VMEM((3 * NSPLIT, 128), jnp.int32),
            pltpu.VMEM((BB, CH), jnp.float32),
            pltpu.VMEM((2, 128, CH), jnp.float32),
            pltpu.VMEM_SHARED((N, CH), jnp.float32),
            pltpu.SemaphoreType.DMA,
            pltpu.SemaphoreType.DMA,
            pltpu.SemaphoreType.DMA,
            pltpu.SemaphoreType.DMA,
        ],
    )
    def k(hcat_ref, pk_ref, tbl_ref, z_ref,
          alo_ref, ahi_ref, pidx, hs, radb, aggs,
          sem_h, sem_r0, sem_r1, sem_s):
        cid = lax.axis_index("c")
        sid = lax.axis_index("s")
        r0 = sid * NROW
        pltpu.sync_copy(z_ref.at[pl.ds(r0, NROW)], aggs.at[pl.ds(r0, NROW)])
        plsc.subcore_barrier()
        hoff = cid * N
        toff = cid * TBL
        sem_r = (sem_r0, sem_r1)

        def batch_body(kk, carry):
            bi = kk * NSUB + sid
            pltpu.sync_copy(pk_ref.at[pl.ds(bi * 3 * NSPLIT, 3 * NSPLIT)],
                            pidx)
            for j in range(NSPLIT):
                for v in range(8):
                    pidx[j, pl.ds(v * 16, 16)] = (
                        pidx[j, pl.ds(v * 16, 16)] + hoff)
                    pidx[2 * NSPLIT + j, pl.ds(v * 16, 16)] = (
                        pidx[2 * NSPLIT + j, pl.ds(v * 16, 16)] + toff)
            h_cps = [pltpu.async_copy(hcat_ref.at[pidx.at[j]],
                                      hs.at[pl.ds(j * 128, 128)], sem_h)
                     for j in range(NSPLIT)]
            r_cps = [None] * NSPLIT
            r_cps[0] = pltpu.async_copy(tbl_ref.at[pidx.at[2 * NSPLIT]],
                                        radb.at[0], sem_r0)
            for cp in h_cps:
                cp.wait()
            sc_cps = []
            for j in range(NSPLIT):
                if j + 1 < NSPLIT:
                    r_cps[j + 1] = pltpu.async_copy(
                        tbl_ref.at[pidx.at[2 * NSPLIT + j + 1]],
                        radb.at[(j + 1) % 2], sem_r[(j + 1) % 2])
                r_cps[j].wait()

                def mul_body(e, carry2, _j=j):
                    e2 = _j * 128 + e
                    rs = _j % 2
                    hs[e2, pl.ds(0, 16)] = (
                        hs[e2, pl.ds(0, 16)] * radb[rs, e, pl.ds(0, 16)])
                    hs[e2, pl.ds(16, 16)] = (
                        hs[e2, pl.ds(16, 16)] * radb[rs, e, pl.ds(16, 16)])
                    return carry2

                lax.fori_loop(0, 128, mul_body, 0)
                sc_cps.append(pltpu.async_copy(
                    hs.at[pl.ds(j * 128, 128)],
                    aggs.at[pidx.at[NSPLIT + j]], sem_s, add=True))
            for cp in sc_cps:
                cp.wait()
            return carry

        lax.fori_loop(0, NBATCH // NSUB, batch_body, 0)
        plsc.subcore_barrier()

        @pl.when(cid == 0)
        def _():
            pltpu.sync_copy(aggs.at[pl.ds(r0, NROW)], alo_ref.at[pl.ds(r0, NROW)])

        @pl.when(cid == 1)
        def _():
            pltpu.sync_copy(aggs.at[pl.ds(r0, NROW)], ahi_ref.at[pl.ds(r0, NROW)])

    return k(hcat, pk2d, table2, zrows)


def _table_body(wrad_ref, out_ref):
    i = pl.program_id(0)
    t = (lax.broadcasted_iota(jnp.int32, (BT, 1), 0) + i * BT
         ).astype(jnp.float32)
    r = (t + 0.5) * (CUTOFF / TBL)
    rs = r + 1e-9
    theta = r * (_PI / CUTOFF)
    su, cu = _sin_cos(theta - _PI / 2)
    s_n = cu
    two_c = -(su + su)
    x = r * (1.0 / CUTOFF)
    x2 = x * x
    x5 = x2 * x2 * x
    env = 1.0 + x5 * (-21.0 + 35.0 * x - 15.0 * x2)
    coef = 0.6324555320336759 * env / rs
    cols = []
    s_prev = jnp.zeros((BT, 1), jnp.float32)
    for _ in range(NB):
        cols.append(coef * s_n)
        s_next = two_c * s_n - s_prev
        s_prev = s_n
        s_n = s_next
    rb = jnp.concatenate(cols, axis=1)
    g = jnp.dot(rb, wrad_ref[...], preferred_element_type=jnp.float32)
    rad = _silu(g)
    out_ref[0] = rad[:, :CH]
    out_ref[1] = rad[:, CH:]


def _tc_table(W_rad):
    return pl.pallas_call(
        _table_body,
        grid=(TBL // BT,),
        in_specs=[pl.BlockSpec((NB, C), lambda i: (0, 0))],
        out_specs=pl.BlockSpec((2, BT, CH), lambda i: (0, i, 0)),
        out_shape=jax.ShapeDtypeStruct((2, TBL, CH), jnp.float32),
    )(W_rad)


def _embed_body(sp_ref, we_ref, h2_ref):
    sp = sp_ref[0, 0, :]
    oh = (sp[:, None] == lax.broadcasted_iota(jnp.int32, (BN, NELEM), 1)
          ).astype(jnp.float32)
    h0 = jnp.dot(oh, we_ref[...], preferred_element_type=jnp.float32)
    h2_ref[0] = h0[:, :CH]
    h2_ref[1] = h0[:, CH:]


def _tc_embed(species_r, W_embed):
    return pl.pallas_call(
        _embed_body,
        grid=(GRID_N,),
        in_specs=[
            pl.BlockSpec((1, 1, BN), lambda i: (i, 0, 0)),
            pl.BlockSpec((NELEM, C), lambda i: (0, 0)),
        ],
        out_specs=pl.BlockSpec((2, BN, CH), lambda i: (0, i, 0)),
        out_shape=jax.ShapeDtypeStruct((2, N, CH), jnp.float32),
    )(species_r, W_embed)


def _layer_body(alo_ref, ahi_ref, h2_ref, wi_ref, ws_ref, wr_ref,
                n2_ref, e_ref):
    agg = jnp.concatenate([alo_ref[...], ahi_ref[...]], axis=1) * (1.0 / AVG_NEIGH)
    h = jnp.concatenate([h2_ref[0], h2_ref[1]], axis=1)
    z = (jnp.dot(agg, wi_ref[...], preferred_element_type=jnp.float32)
         + jnp.dot(h, ws_ref[...], preferred_element_type=jnp.float32))
    hn = _silu(z)
    n2_ref[0] = hn[:, :CH]
    n2_ref[1] = hn[:, CH:]
    e_ref[...] = jnp.dot(hn, wr_ref[...], preferred_element_type=jnp.float32)


def _tc_layer(agg_lo, agg_hi, h2, W_inter, W_skip, w_read):
    return pl.pallas_call(
        _layer_body,
        grid=(GRID_N,),
        in_specs=[
            pl.BlockSpec((BN, CH), lambda i: (i, 0)),
            pl.BlockSpec((BN, CH), lambda i: (i, 0)),
            pl.BlockSpec((2, BN, CH), lambda i: (0, i, 0)),
            pl.BlockSpec((C, C), lambda i: (0, 0)),
            pl.BlockSpec((C, C), lambda i: (0, 0)),
            pl.BlockSpec((C, 1), lambda i: (0, 0)),
        ],
        out_specs=[
            pl.BlockSpec((2, BN, CH), lambda i: (0, i, 0)),
            pl.BlockSpec((BN, 1), lambda i: (i, 0)),
        ],
        out_shape=[
            jax.ShapeDtypeStruct((2, N, CH), jnp.float32),
            jax.ShapeDtypeStruct((N, 1), jnp.float32),
        ],
    )(agg_lo, agg_hi, h2, W_inter, W_skip, w_read)


def _final_body(alo_ref, ahi_ref, h2_ref, wi_ref, ws_ref, wr_ref,
                e0_ref, sp_ref, ae_ref, sc_ref, sh_ref, out_ref):
    agg = jnp.concatenate([alo_ref[...], ahi_ref[...]], axis=1) * (1.0 / AVG_NEIGH)
    h = jnp.concatenate([h2_ref[0], h2_ref[1]], axis=1)
    z = (jnp.dot(agg, wi_ref[...], preferred_element_type=jnp.float32)
         + jnp.dot(h, ws_ref[...], preferred_element_type=jnp.float32))
    hn = _silu(z)
    e1 = jnp.dot(hn, wr_ref[...], preferred_element_type=jnp.float32)
    sp = sp_ref[0, 0, :]
    oh = (sp[:, None] == lax.broadcasted_iota(jnp.int32, (BN, NELEM), 1)
          ).astype(jnp.float32)
    e_base = jnp.dot(oh, ae_ref[...], preferred_element_type=jnp.float32)
    e_read = e0_ref[...] + e1
    out_ref[...] = e_base + sc_ref[0, 0] * e_read + sh_ref[0, 0]


def _tc_final(agg_lo, agg_hi, h2, W_inter, W_skip, w_read, e0,
              species_r, ae2, scale2, shift2):
    return pl.pallas_call(
        _final_body,
        grid=(GRID_N,),
        in_specs=[
            pl.BlockSpec((BN, CH), lambda i: (i, 0)),
            pl.BlockSpec((BN, CH), lambda i: (i, 0)),
            pl.BlockSpec((2, BN, CH), lambda i: (0, i, 0)),
            pl.BlockSpec((C, C), lambda i: (0, 0)),
            pl.BlockSpec((C, C), lambda i: (0, 0)),
            pl.BlockSpec((C, 1), lambda i: (0, 0)),
            pl.BlockSpec((BN, 1), lambda i: (i, 0)),
            pl.BlockSpec((1, 1, BN), lambda i: (i, 0, 0)),
            pl.BlockSpec((NELEM, 1), lambda i: (0, 0)),
            pl.BlockSpec((1, 1), lambda i: (0, 0)),
            pl.BlockSpec((1, 1), lambda i: (0, 0)),
        ],
        out_specs=pl.BlockSpec((BN, 1), lambda i: (i, 0)),
        out_shape=jax.ShapeDtypeStruct((N, 1), jnp.float32),
    )(agg_lo, agg_hi, h2, W_inter, W_skip, w_read, e0,
      species_r, ae2, scale2, shift2)


def kernel(positions, species, edge_index, W_embed, W_rad0, W_rad1,
           W_inter0, W_inter1, W_skip0, W_skip1, w_read0, w_read1,
           atomic_energies, scale, shift):
    src2d = jnp.pad(edge_index[0].astype(jnp.int32),
                    (0, EPAD - E)).reshape(EPAD // 128, 128)
    dst2d = jnp.pad(edge_index[1].astype(jnp.int32),
                    (0, EPAD - E)).reshape(EPAD // 128, 128)
    pos_pad = jnp.pad(positions, ((0, 0), (0, 13)))
    zrows = jnp.zeros((N, CH), jnp.float32)
    species_r = species.astype(jnp.int32).reshape(GRID_N, 1, BN)
    ae2 = atomic_energies.reshape(NELEM, 1)
    scale2 = scale.reshape(1, 1)
    shift2 = shift.reshape(1, 1)

    ridx = _rb_sc(pos_pad, src2d, dst2d)
    pk2d = jnp.concatenate(
        [src2d.reshape(NBATCH, NSPLIT, 128),
         dst2d.reshape(NBATCH, NSPLIT, 128),
         ridx.reshape(NBATCH, NSPLIT, 128)], axis=1,
    ).reshape(NBATCH * 3 * NSPLIT, 128)
    tbl0 = _tc_table(W_rad0).reshape(2 * TBL, CH)
    tbl1 = _tc_table(W_rad1).reshape(2 * TBL, CH)
    h2 = _tc_embed(species_r, W_embed)

    a0lo, a0hi = _edge_sc(h2.reshape(2 * N, CH), pk2d, tbl0, zrows)
    h2n, e0 = _tc_layer(a0lo, a0hi, h2, W_inter0, W_skip0, w_read0)

    a1lo, a1hi = _edge_sc(h2n.reshape(2 * N, CH), pk2d, tbl1, zrows)
    out = _tc_final(a1lo, a1hi, h2n, W_inter1, W_skip1, w_read1, e0,
                    species_r, ae2, scale2, shift2)
    return out[:, 0]
